# baseline (speedup 1.0000x reference)
import jax
import jax.numpy as jnp
from jax.experimental import pallas as pl


def _final_body(ge_ref, wm_ref, bm_ref, out_ref):
    out_ref[...] = jax.nn.relu(
        jnp.dot(ge_ref[...], wm_ref[...], preferred_element_type=jnp.float32)
        + bm_ref[...][None, :]
    )


def kernel(x, edge_index, edge_attr, batch, graph_fea, embed, W1, b1, W2, b2, Wg, bg, Wa, Wm, bm):
    n = x.shape[0]
    g = graph_fea.shape[0]
    loop = jnp.arange(n)
    s = jnp.concatenate([edge_index[0], loop])
    d = jnp.concatenate([edge_index[1], loop])
    ew = jnp.concatenate([edge_attr, jnp.ones((n,), edge_attr.dtype)])
    deg = jax.ops.segment_sum(ew, d, num_segments=n)
    deg_safe = jnp.where(deg > 0, deg, 1.0)
    dis = jnp.where(deg > 0, jax.lax.rsqrt(deg_safe), 0.0)
    norm = dis[s] * ew * dis[d]

    def gcn(h, W, b):
        h = h @ W
        msg = norm[:, None] * h[s]
        return jax.ops.segment_sum(msg, d, num_segments=n) + b

    h = embed[x]
    h = jax.nn.relu(gcn(h, W1, b1))
    h = gcn(h, W2, b2)
    m = (graph_fea @ Wg + bg)[batch]
    logits = jnp.tanh(jnp.concatenate([h, m], axis=-1)) @ Wa
    smax = jax.ops.segment_max(logits, batch, num_segments=g)
    ex = jnp.exp(logits - smax[batch])
    ssum = jax.ops.segment_sum(ex, batch, num_segments=g)
    att = ex / ssum[batch]
    xw = 0.2 * h * att + 0.8 * m
    ge = jax.ops.segment_sum(xw, batch, num_segments=g)

    out = pl.pallas_call(
        _final_body,
        out_shape=jax.ShapeDtypeStruct((g, Wm.shape[1]), jnp.float32),
    )(ge, Wm, bm)
    return out


# speedup vs baseline: 4.2816x; 4.2816x over previous
import functools

import jax
import jax.numpy as jnp
from jax import lax
from jax.experimental import pallas as pl
from jax.experimental.pallas import tpu as pltpu
from jax.experimental.pallas import tpu_sc as plsc

F32 = jnp.float32
I32 = jnp.int32

NW = 32
NPW = 320
NPAD = NW * NPW
EPAD = 163840
EPW = EPAD // NW
CE = 2560
K = 32
V = 128


def _wid():
    return lax.axis_index("s") * 2 + lax.axis_index("c")


def _deg_body(d_hbm, ew_hbm, out_hbm, dbuf, ewbuf, acc):
    w = _wid()
    base = w * EPW
    pltpu.sync_copy(d_hbm.at[pl.ds(base, EPW)], dbuf)
    pltpu.sync_copy(ew_hbm.at[pl.ds(base, EPW)], ewbuf)

    def zb(i, _):
        acc[pl.ds(i * 16, 16)] = jnp.zeros((16,), F32)
        return 0

    lax.fori_loop(0, NPAD // 16, zb, 0)

    def eb(g, _):
        dv = dbuf[pl.ds(g * 16, 16)]
        ev = ewbuf[pl.ds(g * 16, 16)]
        plsc.addupdate_scatter(acc, [dv], ev)
        return 0

    lax.fori_loop(0, EPW // 16, eb, 0)
    pltpu.sync_copy(acc, out_hbm.at[w])


def _nk_body(s_hbm, d_hbm, ew_hbm, x_hbm, dis_hbm, norm_hbm, key_hbm,
             sbuf, dbuf, ewbuf, xv, disv, nbuf, kbuf):
    w = _wid()
    base = w * EPW
    pltpu.sync_copy(x_hbm, xv)
    pltpu.sync_copy(dis_hbm, disv)
    pltpu.sync_copy(s_hbm.at[pl.ds(base, EPW)], sbuf)
    pltpu.sync_copy(d_hbm.at[pl.ds(base, EPW)], dbuf)
    pltpu.sync_copy(ew_hbm.at[pl.ds(base, EPW)], ewbuf)

    def eb(g, _):
        sv = sbuf[pl.ds(g * 16, 16)]
        dv = dbuf[pl.ds(g * 16, 16)]
        ev = ewbuf[pl.ds(g * 16, 16)]
        dss = plsc.load_gather(disv, [sv])
        dsd = plsc.load_gather(disv, [dv])
        xs = plsc.load_gather(xv, [sv])
        nbuf[pl.ds(g * 16, 16)] = dss * ev * dsd
        kbuf[pl.ds(g * 16, 16)] = dv * V + xs
        return 0

    lax.fori_loop(0, EPW // 16, eb, 0)
    pltpu.sync_copy(nbuf, norm_hbm.at[pl.ds(base, EPW)])
    pltpu.sync_copy(kbuf, key_hbm.at[pl.ds(base, EPW)])


def _c_body(key_hbm, norm_hbm, x_hbm, dis_hbm, c_hbm, kbuf, nbuf, xloc, disloc, cacc):
    w = _wid()
    lo = w * NPW * V

    def zb(r, _):
        for f in range(V // 16):
            cacc[r, pl.ds(f * 16, 16)] = jnp.zeros((16,), F32)
        return 0

    lax.fori_loop(0, NPW, zb, 0)

    def chunk(ci, _):
        pltpu.sync_copy(key_hbm.at[pl.ds(ci * CE, CE)], kbuf)
        pltpu.sync_copy(norm_hbm.at[pl.ds(ci * CE, CE)], nbuf)

        def eb(g, _):
            kv = kbuf[pl.ds(g * 16, 16)] - lo
            nv = nbuf[pl.ds(g * 16, 16)]
            m = (kv >= 0) & (kv < NPW * V)
            plsc.addupdate_scatter(cacc, [kv >> 7, kv & (V - 1)], nv, mask=m)
            return 0

        lax.fori_loop(0, CE // 16, eb, 0)
        return 0

    lax.fori_loop(0, EPAD // CE, chunk, 0)

    pltpu.sync_copy(x_hbm.at[pl.ds(w * NPW, NPW)], xloc)
    pltpu.sync_copy(dis_hbm.at[pl.ds(w * NPW, NPW)], disloc)

    def sb(g, _):
        xv = xloc[pl.ds(g * 16, 16)]
        dv = disloc[pl.ds(g * 16, 16)]
        rows = lax.iota(I32, 16) + g * 16
        plsc.addupdate_scatter(cacc, [rows, xv], dv * dv)
        return 0

    lax.fori_loop(0, NPW // 16, sb, 0)
    pltpu.sync_copy(cacc, c_hbm.at[pl.ds(w * NPW, NPW)])


def _q_body(d_hbm, s_hbm, norm_hbm, h1_hbm, q_hbm,
            dbuf, sbuf, nbuf, hs, hd, hn, rows, acc, sems):
    w = _wid()
    nlo = w * NPW

    def zb(r, _):
        for f in range(16):
            acc[r, pl.ds(f * 16, 16)] = jnp.zeros((16,), F32)
        return 0

    lax.fori_loop(0, NPW, zb, 0)

    def chunk(ci, _):
        pltpu.sync_copy(d_hbm.at[pl.ds(ci * CE, CE)], dbuf)
        pltpu.sync_copy(s_hbm.at[pl.ds(ci * CE, CE)], sbuf)
        pltpu.sync_copy(norm_hbm.at[pl.ds(ci * CE, CE)], nbuf)

        def eb(g, cnt):
            dv = dbuf[pl.ds(g * 16, 16)] - nlo
            sv = sbuf[pl.ds(g * 16, 16)]
            nv = nbuf[pl.ds(g * 16, 16)]
            m = (dv >= 0) & (dv < NPW)
            mi = m.astype(I32)
            pos = cnt + plsc.cumsum(mi) - 1
            plsc.store_scatter(hs, [pos], sv, mask=m)
            plsc.store_scatter(hd, [pos], dv, mask=m)
            plsc.store_scatter(hn, [pos], nv, mask=m)
            return cnt + jnp.sum(mi)

        cnt = lax.fori_loop(0, CE // 16, eb, 0)
        zi = jnp.zeros((16,), I32)
        zf = jnp.zeros((16,), F32)
        for t in range(3):
            hs[pl.ds(cnt + t * 16, 16)] = zi
            hd[pl.ds(cnt + t * 16, 16)] = zi
            hn[pl.ds(cnt + t * 16, 16)] = zf

        nb = (cnt + K - 1) // K

        def issue(b):
            slot = lax.rem(b, 3)
            pltpu.make_async_copy(
                h1_hbm.at[hs.at[pl.ds(b * K, K)]], rows.at[slot], sems.at[slot]
            ).start()

        @pl.when(nb > 0)
        def _():
            issue(0)

        def bb(b, _):
            slot = lax.rem(b, 3)

            @pl.when(b + 1 < nb)
            def _():
                issue(b + 1)

            pltpu.make_async_copy(
                h1_hbm.at[hs.at[pl.ds(b * K, K)]], rows.at[slot], sems.at[slot]
            ).wait()
            nvecs = [hn[pl.ds(b * K + 16 * t, 16)] for t in range(K // 16)]
            dvecs = [hd[pl.ds(b * K + 16 * t, 16)] for t in range(K // 16)]
            for j in range(K):
                nj = nvecs[j // 16][j % 16]
                dj = dvecs[j // 16][j % 16]
                for f in range(16):
                    acc[dj, pl.ds(f * 16, 16)] = (
                        acc[dj, pl.ds(f * 16, 16)] + nj * rows[slot, j, pl.ds(f * 16, 16)]
                    )
            return 0

        lax.fori_loop(0, nb, bb, 0)
        return 0

    lax.fori_loop(0, EPAD // CE, chunk, 0)
    pltpu.sync_copy(acc, q_hbm.at[pl.ds(nlo, NPW)])


def _pre_body(degp_ref, embed_ref, w1_ref, gf_ref, wg_ref, bg_ref, wa2_ref,
              dis_ref, inv_ref, ew_ref, mrow_ref, la2_ref):
    deg = jnp.sum(degp_ref[...], axis=0) + 1.0
    dis_ref[...] = lax.rsqrt(deg)
    inv_ref[...] = 1.0 / deg
    ew_ref[...] = jnp.dot(embed_ref[...], w1_ref[...], preferred_element_type=F32)
    mrow = jnp.dot(gf_ref[...], wg_ref[...], preferred_element_type=F32) + bg_ref[...][None, :]
    mrow_ref[...] = mrow
    la2_ref[...] = jnp.dot(jnp.tanh(mrow), wa2_ref[...], preferred_element_type=F32)


def _h1_body(c_ref, ew_ref, b1_ref, h1_ref):
    h1_ref[...] = jax.nn.relu(
        jnp.dot(c_ref[...], ew_ref[...], preferred_element_type=F32) + b1_ref[...][None, :]
    )


def _tail_body(q_ref, h1_ref, inv_ref, bat_ref, w2_ref, b2_ref, wa1_ref, la2_ref,
               shex_ref, sex_ref, scnt_ref):
    i = pl.program_id(0)
    qf = q_ref[...] + inv_ref[...] * h1_ref[...]
    h2 = jnp.dot(qf, w2_ref[...], preferred_element_type=F32) + b2_ref[...][None, :]
    th = jnp.tanh(h2)
    oh = (bat_ref[...] == lax.broadcasted_iota(I32, (1, 64), 1)).astype(F32)
    lg = (jnp.dot(th, wa1_ref[...], preferred_element_type=F32)
          + jnp.dot(oh, la2_ref[...], preferred_element_type=F32))
    exv = jnp.exp(lg)
    dn = (((0,), (0,)), ((), ()))
    p_shex = lax.dot_general(oh, h2 * exv, dn, preferred_element_type=F32)
    p_sex = lax.dot_general(oh, exv, dn, preferred_element_type=F32)
    p_scnt = lax.dot_general(oh, jnp.ones_like(exv), dn, preferred_element_type=F32)

    @pl.when(i == 0)
    def _():
        shex_ref[...] = jnp.zeros_like(shex_ref)
        sex_ref[...] = jnp.zeros_like(sex_ref)
        scnt_ref[...] = jnp.zeros_like(scnt_ref)

    shex_ref[...] += p_shex
    sex_ref[...] += p_sex
    scnt_ref[...] += p_scnt


def _fin_body(shex_ref, sex_ref, scnt_ref, mrow_ref, wm_ref, bm_ref, out_ref):
    sex = sex_ref[...]
    ssafe = jnp.where(sex > 0, sex, 1.0)
    ge = 0.2 * shex_ref[...] / ssafe + 0.8 * scnt_ref[...] * mrow_ref[...]
    out_ref[...] = jax.nn.relu(
        jnp.dot(ge, wm_ref[...], preferred_element_type=F32) + bm_ref[...][None, :]
    )


def kernel(x, edge_index, edge_attr, batch, graph_fea, embed, W1, b1, W2, b2,
           Wg, bg, Wa, Wm, bm):
    n = x.shape[0]
    e = edge_index.shape[1]
    g = graph_fea.shape[0]
    h = embed.shape[1]

    ep = EPAD - e
    s_pad = jnp.concatenate([edge_index[0], jnp.zeros((ep,), I32)])
    d_pad = jnp.concatenate([edge_index[1], jnp.full((ep,), n, I32)])
    ew_pad = jnp.concatenate([edge_attr, jnp.zeros((ep,), F32)])
    x_pad = jnp.concatenate([x.astype(I32), jnp.zeros((NPAD - n,), I32)])
    bat_pad = jnp.concatenate([batch.astype(I32), jnp.full((NPAD - n,), g, I32)])
    bat_pad = bat_pad.reshape(NPAD, 1)
    Wa1 = Wa[:h]
    Wa2 = Wa[h:]

    mesh = plsc.VectorSubcoreMesh(core_axis_name="c", subcore_axis_name="s")

    deg_parts = pl.kernel(
        _deg_body,
        out_type=jax.ShapeDtypeStruct((NW, NPAD), F32),
        mesh=mesh,
        compiler_params=pltpu.CompilerParams(needs_layout_passes=False),
        scratch_types=[
            pltpu.VMEM((EPW,), I32),
            pltpu.VMEM((EPW,), F32),
            pltpu.VMEM((NPAD,), F32),
        ],
    )(d_pad, ew_pad)

    dis3, inv3, EW, mrow, la2 = pl.pallas_call(
        _pre_body,
        out_shape=(
            jax.ShapeDtypeStruct((NPAD // 128, 128), F32),
            jax.ShapeDtypeStruct((NPAD // 128, 128), F32),
            jax.ShapeDtypeStruct((V, h // 2), F32),
            jax.ShapeDtypeStruct((g, h), F32),
            jax.ShapeDtypeStruct((g, 1), F32),
        ),
    )(deg_parts.reshape(NW, NPAD // 128, 128), embed, W1, graph_fea, Wg, bg, Wa2)
    dis = dis3.reshape(NPAD)

    norm, key = pl.kernel(
        _nk_body,
        out_type=(
            jax.ShapeDtypeStruct((EPAD,), F32),
            jax.ShapeDtypeStruct((EPAD,), I32),
        ),
        mesh=mesh,
        compiler_params=pltpu.CompilerParams(needs_layout_passes=False),
        scratch_types=[
            pltpu.VMEM((EPW,), I32),
            pltpu.VMEM((EPW,), I32),
            pltpu.VMEM((EPW,), F32),
            pltpu.VMEM((NPAD,), I32),
            pltpu.VMEM((NPAD,), F32),
            pltpu.VMEM((EPW,), F32),
            pltpu.VMEM((EPW,), I32),
        ],
    )(s_pad, d_pad, ew_pad, x_pad, dis)

    C = pl.kernel(
        _c_body,
        out_type=jax.ShapeDtypeStruct((NPAD, V), F32),
        mesh=mesh,
        compiler_params=pltpu.CompilerParams(needs_layout_passes=False),
        scratch_types=[
            pltpu.VMEM((CE,), I32),
            pltpu.VMEM((CE,), F32),
            pltpu.VMEM((NPW,), I32),
            pltpu.VMEM((NPW,), F32),
            pltpu.VMEM((NPW, V), F32),
        ],
    )(key, norm, x_pad, dis)

    h1 = pl.pallas_call(
        _h1_body,
        out_shape=jax.ShapeDtypeStruct((NPAD, h // 2), F32),
    )(C, EW, b1)

    q = pl.kernel(
        _q_body,
        out_type=jax.ShapeDtypeStruct((NPAD, h // 2), F32),
        mesh=mesh,
        compiler_params=pltpu.CompilerParams(needs_layout_passes=False),
        scratch_types=[
            pltpu.VMEM((CE,), I32),
            pltpu.VMEM((CE,), I32),
            pltpu.VMEM((CE,), F32),
            pltpu.VMEM((CE + 64,), I32),
            pltpu.VMEM((CE + 64,), I32),
            pltpu.VMEM((CE + 64,), F32),
            pltpu.VMEM((3, K, h // 2), F32),
            pltpu.VMEM((NPW, h // 2), F32),
            pltpu.SemaphoreType.DMA((3,)),
        ],
    )(d_pad, s_pad, norm, h1)

    nb_blocks = NPAD // 2048
    shex, sex, scnt = pl.pallas_call(
        _tail_body,
        grid=(nb_blocks,),
        in_specs=[
            pl.BlockSpec((2048, h // 2), lambda i: (i, 0)),
            pl.BlockSpec((2048, h // 2), lambda i: (i, 0)),
            pl.BlockSpec((2048, 1), lambda i: (i, 0)),
            pl.BlockSpec((2048, 1), lambda i: (i, 0)),
            pl.BlockSpec((h // 2, h), lambda i: (0, 0)),
            pl.BlockSpec((h,), lambda i: (0,)),
            pl.BlockSpec((h, 1), lambda i: (0, 0)),
            pl.BlockSpec((g, 1), lambda i: (0, 0)),
        ],
        out_specs=(
            pl.BlockSpec((g, h), lambda i: (0, 0)),
            pl.BlockSpec((g, 1), lambda i: (0, 0)),
            pl.BlockSpec((g, 1), lambda i: (0, 0)),
        ),
        out_shape=(
            jax.ShapeDtypeStruct((g, h), F32),
            jax.ShapeDtypeStruct((g, 1), F32),
            jax.ShapeDtypeStruct((g, 1), F32),
        ),
    )(q, h1, inv3.reshape(NPAD, 1), bat_pad, W2, b2, Wa1, la2)

    out = pl.pallas_call(
        _fin_body,
        out_shape=jax.ShapeDtypeStruct((g, h), F32),
    )(shex, sex, scnt, mrow, Wm, bm)
    return out


# speedup vs baseline: 11.0559x; 2.5822x over previous
import functools

import jax
import jax.numpy as jnp
from jax import lax
from jax.experimental import pallas as pl
from jax.experimental.pallas import tpu as pltpu
from jax.experimental.pallas import tpu_sc as plsc

F32 = jnp.float32
I32 = jnp.int32

NW = 32
NPW = 320
NPAD = NW * NPW
EPAD = 163840
EPW = EPAD // NW
CE = 2560
K = 32
V = 128


def _wid():
    return lax.axis_index("s") * 2 + lax.axis_index("c")


def _deg_body(d_hbm, ew_hbm, out_hbm, dbuf, ewbuf, acc):
    w = _wid()
    base = w * EPW
    pltpu.sync_copy(d_hbm.at[pl.ds(base, EPW)], dbuf)
    pltpu.sync_copy(ew_hbm.at[pl.ds(base, EPW)], ewbuf)

    def zb(i, _):
        acc[pl.ds(i * 16, 16)] = jnp.zeros((16,), F32)
        return 0

    lax.fori_loop(0, NPAD // 16, zb, 0)

    def eb(g, _):
        dv = dbuf[pl.ds(g * 16, 16)]
        ev = ewbuf[pl.ds(g * 16, 16)]
        plsc.addupdate_scatter(acc, [dv], ev)
        return 0

    lax.fori_loop(0, EPW // 16, eb, 0)
    pltpu.sync_copy(acc, out_hbm.at[w])


def _nk_body(s_hbm, d_hbm, ew_hbm, x_hbm, dis_hbm, norm_hbm, key_hbm,
             sbuf, dbuf, ewbuf, xv, disv, nbuf, kbuf):
    w = _wid()
    base = w * EPW
    pltpu.sync_copy(x_hbm, xv)
    pltpu.sync_copy(dis_hbm, disv)
    pltpu.sync_copy(s_hbm.at[pl.ds(base, EPW)], sbuf)
    pltpu.sync_copy(d_hbm.at[pl.ds(base, EPW)], dbuf)
    pltpu.sync_copy(ew_hbm.at[pl.ds(base, EPW)], ewbuf)

    def eb(g, _):
        sv = sbuf[pl.ds(g * 16, 16)]
        dv = dbuf[pl.ds(g * 16, 16)]
        ev = ewbuf[pl.ds(g * 16, 16)]
        dss = plsc.load_gather(disv, [sv])
        dsd = plsc.load_gather(disv, [dv])
        xs = plsc.load_gather(xv, [sv])
        nbuf[pl.ds(g * 16, 16)] = dss * ev * dsd
        kbuf[pl.ds(g * 16, 16)] = dv * V + xs
        return 0

    lax.fori_loop(0, EPW // 16, eb, 0)
    pltpu.sync_copy(nbuf, norm_hbm.at[pl.ds(base, EPW)])
    pltpu.sync_copy(kbuf, key_hbm.at[pl.ds(base, EPW)])


def _c_body(key_hbm, norm_hbm, x_hbm, dis_hbm, c_hbm, kbuf, nbuf, xloc, disloc, cacc):
    w = _wid()
    lo = w * NPW * V

    def zb(r, _):
        for f in range(V // 16):
            cacc[r, pl.ds(f * 16, 16)] = jnp.zeros((16,), F32)
        return 0

    lax.fori_loop(0, NPW, zb, 0)

    def chunk(ci, _):
        pltpu.sync_copy(key_hbm.at[pl.ds(ci * CE, CE)], kbuf)
        pltpu.sync_copy(norm_hbm.at[pl.ds(ci * CE, CE)], nbuf)

        def eb(g, _):
            kv = kbuf[pl.ds(g * 16, 16)] - lo
            m = (kv >= 0) & (kv < NPW * V)
            tot = plsc.all_reduce_population_count(m)[0]

            @pl.when(tot > 0)
            def _():
                nv = nbuf[pl.ds(g * 16, 16)]
                plsc.addupdate_scatter(cacc, [kv >> 7, kv & (V - 1)], nv, mask=m)

            return 0

        lax.fori_loop(0, CE // 16, eb, 0)
        return 0

    lax.fori_loop(0, EPAD // CE, chunk, 0)

    pltpu.sync_copy(x_hbm.at[pl.ds(w * NPW, NPW)], xloc)
    pltpu.sync_copy(dis_hbm.at[pl.ds(w * NPW, NPW)], disloc)

    def sb(g, _):
        xv = xloc[pl.ds(g * 16, 16)]
        dv = disloc[pl.ds(g * 16, 16)]
        rows = lax.iota(I32, 16) + g * 16
        plsc.addupdate_scatter(cacc, [rows, xv], dv * dv)
        return 0

    lax.fori_loop(0, NPW // 16, sb, 0)
    pltpu.sync_copy(cacc, c_hbm.at[pl.ds(w * NPW, NPW)])


def _q_body(d_hbm, s_hbm, norm_hbm, h1_hbm, q_hbm,
            dbuf, sbuf, nbuf, hs, hd, hn, rows, acc, sems):
    w = _wid()
    nlo = w * NPW
    hh = h1_hbm.shape[1]

    def zb(r, _):
        acc[pl.ds(r * 16, 16)] = jnp.zeros((16,), F32)
        return 0

    lax.fori_loop(0, hh * NPW // 16, zb, 0)

    def chunk(ci, _):
        pltpu.sync_copy(d_hbm.at[pl.ds(ci * CE, CE)], dbuf)
        pltpu.sync_copy(s_hbm.at[pl.ds(ci * CE, CE)], sbuf)
        pltpu.sync_copy(norm_hbm.at[pl.ds(ci * CE, CE)], nbuf)

        def eb(g, cnt):
            dv = dbuf[pl.ds(g * 16, 16)] - nlo
            m = (dv >= 0) & (dv < NPW)
            tot = plsc.all_reduce_population_count(m)[0]

            @pl.when(tot > 0)
            def _():
                sv = sbuf[pl.ds(g * 16, 16)]
                nv = nbuf[pl.ds(g * 16, 16)]
                pos = cnt + plsc.cumsum(m.astype(I32)) - 1
                plsc.store_scatter(hs, [pos], sv, mask=m)
                plsc.store_scatter(hd, [pos], dv, mask=m)
                plsc.store_scatter(hn, [pos], nv, mask=m)

            return cnt + tot

        cnt = lax.fori_loop(0, CE // 16, eb, 0)
        zi = jnp.zeros((16,), I32)
        zf = jnp.zeros((16,), F32)
        for t in range(K // 16 + 1):
            hs[pl.ds(cnt + t * 16, 16)] = zi
            hd[pl.ds(cnt + t * 16, 16)] = zi
            hn[pl.ds(cnt + t * 16, 16)] = zf

        nb = (cnt + K - 1) // K

        def issue(b):
            slot = lax.rem(b, 3)
            pltpu.make_async_copy(
                h1_hbm.at[hs.at[pl.ds(b * K, K)]], rows.at[slot], sems.at[slot]
            ).start()

        @pl.when(nb > 0)
        def _():
            issue(0)

        def bb(b, _):
            slot = lax.rem(b, 3)

            @pl.when(b + 1 < nb)
            def _():
                issue(b + 1)

            pltpu.make_async_copy(
                h1_hbm.at[hs.at[pl.ds(b * K, K)]], rows.at[slot], sems.at[slot]
            ).wait()
            rslot = rows.at[slot]
            grp = [
                (
                    hd[pl.ds(b * K + t * 16, 16)],
                    hn[pl.ds(b * K + t * 16, 16)],
                    lax.iota(I32, 16) + t * 16,
                )
                for t in range(K // 16)
            ]

            def cb(c8, _):
                for dc in range(8):
                    c = c8 * 8 + dc
                    cv = jnp.full((16,), c, I32)
                    for dlocv, normv, rowv in grp:
                        vals = plsc.load_gather(rslot, [rowv, cv])
                        plsc.addupdate_scatter(acc, [dlocv + c * NPW], vals * normv)
                return 0

            lax.fori_loop(0, hh // 8, cb, 0)
            return 0

        lax.fori_loop(0, nb, bb, 0)
        return 0

    lax.fori_loop(0, EPAD // CE, chunk, 0)
    pltpu.sync_copy(acc, q_hbm.at[w])


def _pre_body(degp_ref, embed_ref, w1_ref, gf_ref, wg_ref, bg_ref, wa2_ref,
              dis_ref, inv_ref, ew_ref, mrow_ref, la2_ref):
    deg = jnp.sum(degp_ref[...], axis=0) + 1.0
    dis_ref[...] = lax.rsqrt(deg)
    inv_ref[...] = 1.0 / deg
    ew_ref[...] = jnp.dot(embed_ref[...], w1_ref[...], preferred_element_type=F32)
    mrow = jnp.dot(gf_ref[...], wg_ref[...], preferred_element_type=F32) + bg_ref[...][None, :]
    mrow_ref[...] = mrow
    la2_ref[...] = jnp.dot(jnp.tanh(mrow), wa2_ref[...], preferred_element_type=F32)


def _h1_body(c_ref, ew_ref, b1_ref, h1_ref):
    h1_ref[...] = jax.nn.relu(
        jnp.dot(c_ref[...], ew_ref[...], preferred_element_type=F32) + b1_ref[...][None, :]
    )


def _tail_body(qt_ref, h1_ref, inv_ref, bat_ref, w2_ref, b2_ref, wa1_ref, la2_ref,
               shex_ref, sex_ref, scnt_ref):
    i = pl.program_id(0)
    dnt = (((0,), (0,)), ((), ()))
    h2 = (lax.dot_general(qt_ref[...], w2_ref[...], dnt, preferred_element_type=F32)
          + jnp.dot(inv_ref[...] * h1_ref[...], w2_ref[...], preferred_element_type=F32)
          + b2_ref[...][None, :])
    th = jnp.tanh(h2)
    oh = (bat_ref[...] == lax.broadcasted_iota(I32, (1, 64), 1)).astype(F32)
    lg = (jnp.dot(th, wa1_ref[...], preferred_element_type=F32)
          + jnp.dot(oh, la2_ref[...], preferred_element_type=F32))
    exv = jnp.exp(lg)
    dn = (((0,), (0,)), ((), ()))
    p_shex = lax.dot_general(oh, h2 * exv, dn, preferred_element_type=F32)
    p_sex = lax.dot_general(oh, exv, dn, preferred_element_type=F32)
    p_scnt = lax.dot_general(oh, jnp.ones_like(exv), dn, preferred_element_type=F32)

    @pl.when(i == 0)
    def _():
        shex_ref[...] = jnp.zeros_like(shex_ref)
        sex_ref[...] = jnp.zeros_like(sex_ref)
        scnt_ref[...] = jnp.zeros_like(scnt_ref)

    shex_ref[...] += p_shex
    sex_ref[...] += p_sex
    scnt_ref[...] += p_scnt


def _fin_body(shex_ref, sex_ref, scnt_ref, mrow_ref, wm_ref, bm_ref, out_ref):
    sex = sex_ref[...]
    ssafe = jnp.where(sex > 0, sex, 1.0)
    ge = 0.2 * shex_ref[...] / ssafe + 0.8 * scnt_ref[...] * mrow_ref[...]
    out_ref[...] = jax.nn.relu(
        jnp.dot(ge, wm_ref[...], preferred_element_type=F32) + bm_ref[...][None, :]
    )


def kernel(x, edge_index, edge_attr, batch, graph_fea, embed, W1, b1, W2, b2,
           Wg, bg, Wa, Wm, bm):
    n = x.shape[0]
    e = edge_index.shape[1]
    g = graph_fea.shape[0]
    h = embed.shape[1]

    ep = EPAD - e
    s_pad = jnp.concatenate([edge_index[0], jnp.zeros((ep,), I32)])
    d_pad = jnp.concatenate([edge_index[1], jnp.full((ep,), n, I32)])
    ew_pad = jnp.concatenate([edge_attr, jnp.zeros((ep,), F32)])
    x_pad = jnp.concatenate([x.astype(I32), jnp.zeros((NPAD - n,), I32)])
    bat_pad = jnp.concatenate([batch.astype(I32), jnp.full((NPAD - n,), g, I32)])
    bat_pad = bat_pad.reshape(NPAD, 1)
    Wa1 = Wa[:h]
    Wa2 = Wa[h:]

    mesh = plsc.VectorSubcoreMesh(core_axis_name="c", subcore_axis_name="s")

    deg_parts = pl.kernel(
        _deg_body,
        out_type=jax.ShapeDtypeStruct((NW, NPAD), F32),
        mesh=mesh,
        compiler_params=pltpu.CompilerParams(needs_layout_passes=False),
        scratch_types=[
            pltpu.VMEM((EPW,), I32),
            pltpu.VMEM((EPW,), F32),
            pltpu.VMEM((NPAD,), F32),
        ],
    )(d_pad, ew_pad)

    dis3, inv3, EW, mrow, la2 = pl.pallas_call(
        _pre_body,
        out_shape=(
            jax.ShapeDtypeStruct((NPAD // 128, 128), F32),
            jax.ShapeDtypeStruct((NPAD // 128, 128), F32),
            jax.ShapeDtypeStruct((V, h // 2), F32),
            jax.ShapeDtypeStruct((g, h), F32),
            jax.ShapeDtypeStruct((g, 1), F32),
        ),
    )(deg_parts.reshape(NW, NPAD // 128, 128), embed, W1, graph_fea, Wg, bg, Wa2)
    dis = dis3.reshape(NPAD)

    norm, key = pl.kernel(
        _nk_body,
        out_type=(
            jax.ShapeDtypeStruct((EPAD,), F32),
            jax.ShapeDtypeStruct((EPAD,), I32),
        ),
        mesh=mesh,
        compiler_params=pltpu.CompilerParams(needs_layout_passes=False),
        scratch_types=[
            pltpu.VMEM((EPW,), I32),
            pltpu.VMEM((EPW,), I32),
            pltpu.VMEM((EPW,), F32),
            pltpu.VMEM((NPAD,), I32),
            pltpu.VMEM((NPAD,), F32),
            pltpu.VMEM((EPW,), F32),
            pltpu.VMEM((EPW,), I32),
        ],
    )(s_pad, d_pad, ew_pad, x_pad, dis)

    C = pl.kernel(
        _c_body,
        out_type=jax.ShapeDtypeStruct((NPAD, V), F32),
        mesh=mesh,
        compiler_params=pltpu.CompilerParams(needs_layout_passes=False),
        scratch_types=[
            pltpu.VMEM((CE,), I32),
            pltpu.VMEM((CE,), F32),
            pltpu.VMEM((NPW,), I32),
            pltpu.VMEM((NPW,), F32),
            pltpu.VMEM((NPW, V), F32),
        ],
    )(key, norm, x_pad, dis)

    h1 = pl.pallas_call(
        _h1_body,
        out_shape=jax.ShapeDtypeStruct((NPAD, h // 2), F32),
    )(C, EW, b1)

    q3 = pl.kernel(
        _q_body,
        out_type=jax.ShapeDtypeStruct((NW, (h // 2) * NPW), F32),
        mesh=mesh,
        compiler_params=pltpu.CompilerParams(needs_layout_passes=False),
        scratch_types=[
            pltpu.VMEM((CE,), I32),
            pltpu.VMEM((CE,), I32),
            pltpu.VMEM((CE,), F32),
            pltpu.VMEM((CE + 64,), I32),
            pltpu.VMEM((CE + 64,), I32),
            pltpu.VMEM((CE + 64,), F32),
            pltpu.VMEM((3, K, h // 2), F32),
            pltpu.VMEM(((h // 2) * NPW,), F32),
            pltpu.SemaphoreType.DMA((3,)),
        ],
    )(d_pad, s_pad, norm, h1)
    qT = jnp.transpose(q3.reshape(NW, h // 2, NPW), (1, 0, 2)).reshape(h // 2, NPAD)

    nb_blocks = NPAD // 2048
    shex, sex, scnt = pl.pallas_call(
        _tail_body,
        grid=(nb_blocks,),
        in_specs=[
            pl.BlockSpec((h // 2, 2048), lambda i: (0, i)),
            pl.BlockSpec((2048, h // 2), lambda i: (i, 0)),
            pl.BlockSpec((2048, 1), lambda i: (i, 0)),
            pl.BlockSpec((2048, 1), lambda i: (i, 0)),
            pl.BlockSpec((h // 2, h), lambda i: (0, 0)),
            pl.BlockSpec((h,), lambda i: (0,)),
            pl.BlockSpec((h, 1), lambda i: (0, 0)),
            pl.BlockSpec((g, 1), lambda i: (0, 0)),
        ],
        out_specs=(
            pl.BlockSpec((g, h), lambda i: (0, 0)),
            pl.BlockSpec((g, 1), lambda i: (0, 0)),
            pl.BlockSpec((g, 1), lambda i: (0, 0)),
        ),
        out_shape=(
            jax.ShapeDtypeStruct((g, h), F32),
            jax.ShapeDtypeStruct((g, 1), F32),
            jax.ShapeDtypeStruct((g, 1), F32),
        ),
    )(qT, h1, inv3.reshape(NPAD, 1), bat_pad, W2, b2, Wa1, la2)

    out = pl.pallas_call(
        _fin_body,
        out_shape=jax.ShapeDtypeStruct((g, h), F32),
    )(shex, sex, scnt, mrow, Wm, bm)
    return out
